# baseline (speedup 1.0000x reference)
import functools

import jax
import jax.numpy as jnp
from jax import lax
from jax.experimental import pallas as pl
from jax.experimental.pallas import tpu as pltpu
from jax.experimental.pallas import tpu_sc as plsc

_N = 10000
_M = 10000
_K = 16
_D = 128
_P = _N * _K
_NW = 32
_CHUNK = 5120
_PPAD = _NW * _CHUNK
_BP = 8192
_BN = _BP // _K
_GRID = -(-_P // _BP)
_BPF = 8192
_BNF = _BPF // _K
_GRIDF = -(-_P // _BPF)
_EPS = 1e-6
_MM = (((1,), (0,)), ((), ()))


def _sc_gather(raw2d, idx_pad):
    mesh = plsc.VectorSubcoreMesh(core_axis_name="c", subcore_axis_name="s")

    @functools.partial(
        pl.kernel,
        mesh=mesh,
        out_type=jax.ShapeDtypeStruct((6, _PPAD), jnp.float32),
        compiler_params=pltpu.CompilerParams(needs_layout_passes=False),
        scratch_types=[
            pltpu.VMEM((_M * 6,), jnp.float32),
            pltpu.VMEM((_CHUNK,), jnp.int32),
            pltpu.VMEM((6, _CHUNK), jnp.float32),
        ],
    )
    def gather_kernel(raw_hbm, idx_hbm, out_hbm, table_v, idx_v, nb_v):
        wid = lax.axis_index("s") * 2 + lax.axis_index("c")
        base = wid * _CHUNK
        pltpu.sync_copy(raw_hbm, table_v)
        pltpu.sync_copy(idx_hbm.at[pl.ds(base, _CHUNK)], idx_v)

        @plsc.parallel_loop(0, _CHUNK, 16, unroll=16)
        def _(off):
            rows = idx_v[pl.ds(off, 16)] * 6
            for ch in range(6):
                nb_v[ch, pl.ds(off, 16)] = plsc.load_gather(table_v, [rows + ch])
        pltpu.sync_copy(nb_v, out_hbm.at[:, pl.ds(base, _CHUNK)])

    return gather_kernel(raw2d, idx_pad)


def _valid_mask(i):
    lane = lax.broadcasted_iota(jnp.int32, (1, _BP), 1)
    return (i * _BP + lane) < _P


def _split_dot_onehot(x, e):
    hi = x.astype(jnp.bfloat16).astype(jnp.float32)
    lo = x - hi
    return (lax.dot_general(hi, e, _MM, preferred_element_type=jnp.float32)
            + lax.dot_general(lo, e, _MM, preferred_element_type=jnp.float32))


def _split3_dot(a, b, dims):
    ahi = a.astype(jnp.bfloat16).astype(jnp.float32)
    alo = a - ahi
    bhi = b.astype(jnp.bfloat16).astype(jnp.float32)
    blo = b - bhi

    def d(x, y):
        return lax.dot_general(x, y, dims, preferred_element_type=jnp.float32)

    return d(ahi, bhi) + d(ahi, blo) + d(alo, bhi)


def _stats1(nbT, coordsT, we, wn, wd, wc):
    def kern(nb_ref, ct_ref, we_ref, wn_ref, wd_ref, wc_ref,
             y1_ref, s1_ref, s2_ref, e_ref):
        i = pl.program_id(0)

        @pl.when(i == 0)
        def _():
            r = lax.broadcasted_iota(jnp.int32, (_BN, _BP), 0)
            c = lax.broadcasted_iota(jnp.int32, (_BN, _BP), 1)
            e_ref[...] = (r == (c // _K)).astype(jnp.float32)
            s1_ref[...] = jnp.zeros_like(s1_ref)
            s2_ref[...] = jnp.zeros_like(s2_ref)

        lane_p = lax.broadcasted_iota(jnp.int32, (1, _BP), 1)
        nb = jnp.where(i * _BP + lane_p < _P, nb_ref[...], 0.0)
        lane_n = lax.broadcasted_iota(jnp.int32, (1, _BN), 1)
        ct = jnp.where(i * _BN + lane_n < _N, ct_ref[...], 0.0)

        ext = _split_dot_onehot(ct, e_ref[...])
        diff = ext - nb
        d3 = diff[0:3, :]
        c3 = diff[3:6, :]
        dist = jnp.sqrt(jnp.sum(d3 * d3, axis=0, keepdims=True))
        cdif = jnp.sqrt(jnp.sum(c3 * c3, axis=0, keepdims=True))
        y1 = (_split3_dot(we_ref[...], ext, _MM)
              + _split3_dot(wn_ref[...], nb, _MM)
              + wd_ref[...] * dist + wc_ref[...] * cdif)
        y1_ref[...] = y1
        valid = _valid_mask(i)
        s1_ref[...] += jnp.sum(jnp.where(valid, y1, 0.0), axis=1, keepdims=True)
        s2_ref[...] += jnp.sum(jnp.where(valid, y1 * y1, 0.0), axis=1,
                               keepdims=True)

    return pl.pallas_call(
        kern,
        grid=(_GRID,),
        in_specs=[
            pl.BlockSpec((6, _BP), lambda i: (0, i)),
            pl.BlockSpec((6, _BN), lambda i: (0, i)),
            pl.BlockSpec((16, 6), lambda i: (0, 0)),
            pl.BlockSpec((16, 6), lambda i: (0, 0)),
            pl.BlockSpec((16, 1), lambda i: (0, 0)),
            pl.BlockSpec((16, 1), lambda i: (0, 0)),
        ],
        out_specs=[
            pl.BlockSpec((16, _BP), lambda i: (0, i)),
            pl.BlockSpec((16, 1), lambda i: (0, 0)),
            pl.BlockSpec((16, 1), lambda i: (0, 0)),
        ],
        out_shape=[
            jax.ShapeDtypeStruct((16, _GRID * _BP), jnp.float32),
            jax.ShapeDtypeStruct((16, 1), jnp.float32),
            jax.ShapeDtypeStruct((16, 1), jnp.float32),
        ],
        scratch_shapes=[pltpu.VMEM((_BN, _BP), jnp.float32)],
    )(nbT, coordsT, we, wn, wd, wc)


def _stats2(y1hbm, a1, c1):
    def kern(y1_ref, a1_ref, c1_ref, hs_ref, hh_ref):
        i = pl.program_id(0)

        @pl.when(i == 0)
        def _():
            hs_ref[...] = jnp.zeros_like(hs_ref)
            hh_ref[...] = jnp.zeros_like(hh_ref)

        h = jnp.maximum(a1_ref[...] * y1_ref[...] + c1_ref[...], 0.0)
        hm = jnp.where(_valid_mask(i), h, 0.0)
        hs_ref[...] += jnp.sum(hm, axis=1, keepdims=True)
        hh_ref[...] += lax.dot_general(hm, h, (((1,), (1,)), ((), ())),
                                       preferred_element_type=jnp.float32)

    return pl.pallas_call(
        kern,
        grid=(_GRID,),
        in_specs=[
            pl.BlockSpec((16, _BP), lambda i: (0, i)),
            pl.BlockSpec((16, 1), lambda i: (0, 0)),
            pl.BlockSpec((16, 1), lambda i: (0, 0)),
        ],
        out_specs=[
            pl.BlockSpec((16, 1), lambda i: (0, 0)),
            pl.BlockSpec((16, 16), lambda i: (0, 0)),
        ],
        out_shape=[
            jax.ShapeDtypeStruct((16, 1), jnp.float32),
            jax.ShapeDtypeStruct((16, 16), jnp.float32),
        ],
    )(y1hbm, a1, c1)


def _final(y1hbm, featsND, a1, c1, a2w, c2r):
    def kern(y1_ref, f_ref, a1_ref, c1_ref, a2w_ref, c2_ref, x_ref, ste_ref):
        h = jnp.maximum(a1_ref[...] * y1_ref[...] + c1_ref[...], 0.0)
        ot = jnp.maximum(
            _split3_dot(h, a2w_ref[...], (((0,), (1,)), ((), ())))
            + c2_ref[...], 0.0)
        ef = jnp.broadcast_to(f_ref[...][:, None, :],
                              (_BNF, _K, _D)).reshape(_BPF, _D)
        ste_ref[...] = ot
        x_ref[:, 0:_D] = ot
        x_ref[:, _D:2 * _D] = ef

    return pl.pallas_call(
        kern,
        grid=(_GRIDF,),
        in_specs=[
            pl.BlockSpec((16, _BPF), lambda i: (0, i)),
            pl.BlockSpec((_BNF, _D), lambda i: (i, 0)),
            pl.BlockSpec((16, 1), lambda i: (0, 0)),
            pl.BlockSpec((16, 1), lambda i: (0, 0)),
            pl.BlockSpec((_D, 16), lambda i: (0, 0)),
            pl.BlockSpec((1, _D), lambda i: (0, 0)),
        ],
        out_specs=[
            pl.BlockSpec((_BPF, 2 * _D), lambda i: (i, 0)),
            pl.BlockSpec((_BPF, _D), lambda i: (i, 0)),
        ],
        out_shape=[
            jax.ShapeDtypeStruct((_P, 2 * _D), jnp.float32),
            jax.ShapeDtypeStruct((_P, _D), jnp.float32),
        ],
    )(y1hbm, featsND, a1, c1, a2w, c2r)


def kernel(coords, raw_points, features, neigh_idx, whether_cal,
           W0, b0, g0, be0, W1, b1, g1, be1):
    f32 = jnp.float32
    idx = neigh_idx.astype(jnp.int32).reshape(_P)
    idx_pad = jnp.concatenate([idx, jnp.zeros((_PPAD - _P,), jnp.int32)])
    rawflat = raw_points.reshape(_M * 6).astype(f32)

    nbT = _sc_gather(rawflat, idx_pad)

    coordsT = jnp.transpose(coords.reshape(_N, 6).astype(f32))
    featsND = jnp.transpose(features.reshape(_D, _N).astype(f32))

    W0f = W0.astype(f32)
    we0 = W0f[:, 0:6] + W0f[:, 12:18]
    wn0 = W0f[:, 6:12] - W0f[:, 12:18]
    wd0 = W0f[:, 18:19]
    wc0 = W0f[:, 19:20]

    y1hbm, s1, s2 = _stats1(nbT, coordsT, we0, wn0, wd0, wc0)
    mean1 = s1[:, 0] / _P
    var1 = s2[:, 0] / _P - mean1 * mean1
    a1 = g0 * lax.rsqrt(var1 + _EPS)
    c1 = be0 - a1 * mean1

    hs, hh = _stats2(y1hbm, a1[:, None], c1[:, None])
    W1f = W1.astype(f32)
    mh = hs[:, 0] / _P
    HH = hh / _P
    mean2 = jnp.dot(W1f, mh, precision=lax.Precision.HIGHEST)
    wHH = jnp.dot(W1f, HH, precision=lax.Precision.HIGHEST)
    var2 = jnp.sum(wHH * W1f, axis=1) - mean2 * mean2
    a2 = g1 * lax.rsqrt(var2 + _EPS)
    c2 = be1 - a2 * mean2
    a2w = a2[:, None] * W1.astype(f32)

    x2, ste2 = _final(y1hbm, featsND, a1[:, None], c1[:, None],
                      a2w, c2[None, :])
    x = jnp.transpose(x2.reshape(1, _N, _K, 2 * _D), (0, 3, 1, 2))
    ste = jnp.transpose(ste2.reshape(1, _N, _K, _D), (0, 3, 1, 2))
    return (x, ste)


# speedup vs baseline: 1.0165x; 1.0165x over previous
import functools

import jax
import jax.numpy as jnp
from jax import lax
from jax.experimental import pallas as pl
from jax.experimental.pallas import tpu as pltpu
from jax.experimental.pallas import tpu_sc as plsc

_N = 10000
_M = 10000
_K = 16
_D = 128
_P = _N * _K
_NW = 32
_CHUNK = 5120
_PPAD = _NW * _CHUNK
_BP = 4096
_BN = _BP // _K
_GRID = -(-_P // _BP)
_BPF = 8192
_BNF = _BPF // _K
_GRIDF = -(-_P // _BPF)
_EPS = 1e-6
_MM = (((1,), (0,)), ((), ()))


def _sc_gather(raw2d, idx_pad):
    mesh = plsc.VectorSubcoreMesh(core_axis_name="c", subcore_axis_name="s")

    @functools.partial(
        pl.kernel,
        mesh=mesh,
        out_type=jax.ShapeDtypeStruct((6, _PPAD), jnp.float32),
        compiler_params=pltpu.CompilerParams(needs_layout_passes=False),
        scratch_types=[
            pltpu.VMEM((_M * 6,), jnp.float32),
            pltpu.VMEM((_CHUNK,), jnp.int32),
            pltpu.VMEM((6, _CHUNK), jnp.float32),
        ],
    )
    def gather_kernel(raw_hbm, idx_hbm, out_hbm, table_v, idx_v, nb_v):
        wid = lax.axis_index("s") * 2 + lax.axis_index("c")
        base = wid * _CHUNK
        pltpu.sync_copy(raw_hbm, table_v)
        pltpu.sync_copy(idx_hbm.at[pl.ds(base, _CHUNK)], idx_v)

        @plsc.parallel_loop(0, _CHUNK, 16, unroll=16)
        def _(off):
            rows = idx_v[pl.ds(off, 16)] * 6
            for ch in range(6):
                nb_v[ch, pl.ds(off, 16)] = plsc.load_gather(table_v, [rows + ch])
        pltpu.sync_copy(nb_v, out_hbm.at[:, pl.ds(base, _CHUNK)])

    return gather_kernel(raw2d, idx_pad)


def _valid_mask(i):
    lane = lax.broadcasted_iota(jnp.int32, (1, _BP), 1)
    return (i * _BP + lane) < _P


def _split_dot_onehot(x, e):
    hi = x.astype(jnp.bfloat16).astype(jnp.float32)
    lo = x - hi
    return (lax.dot_general(hi, e, _MM, preferred_element_type=jnp.float32)
            + lax.dot_general(lo, e, _MM, preferred_element_type=jnp.float32))


def _split3_dot(a, b, dims):
    ahi = a.astype(jnp.bfloat16).astype(jnp.float32)
    alo = a - ahi
    bhi = b.astype(jnp.bfloat16).astype(jnp.float32)
    blo = b - bhi

    def d(x, y):
        return lax.dot_general(x, y, dims, preferred_element_type=jnp.float32)

    return d(ahi, bhi) + d(ahi, blo) + d(alo, bhi)


def _stats1(nbT, coordsT, we, wn, wd, wc):
    def kern(nb_ref, ct_ref, we_ref, wn_ref, wd_ref, wc_ref,
             y1_ref, s1_ref, s2_ref, e_ref):
        i = pl.program_id(0)

        @pl.when(i == 0)
        def _():
            r = lax.broadcasted_iota(jnp.int32, (_BN, _BP), 0)
            c = lax.broadcasted_iota(jnp.int32, (_BN, _BP), 1)
            e_ref[...] = (r == (c // _K)).astype(jnp.float32)
            s1_ref[...] = jnp.zeros_like(s1_ref)
            s2_ref[...] = jnp.zeros_like(s2_ref)

        lane_p = lax.broadcasted_iota(jnp.int32, (1, _BP), 1)
        nb = jnp.where(i * _BP + lane_p < _P, nb_ref[...], 0.0)
        lane_n = lax.broadcasted_iota(jnp.int32, (1, _BN), 1)
        ct = jnp.where(i * _BN + lane_n < _N, ct_ref[...], 0.0)

        ext = _split_dot_onehot(ct, e_ref[...])
        diff = ext - nb
        d3 = diff[0:3, :]
        c3 = diff[3:6, :]
        dist = jnp.sqrt(jnp.sum(d3 * d3, axis=0, keepdims=True))
        cdif = jnp.sqrt(jnp.sum(c3 * c3, axis=0, keepdims=True))
        y1 = (_split3_dot(we_ref[...], ext, _MM)
              + _split3_dot(wn_ref[...], nb, _MM)
              + wd_ref[...] * dist + wc_ref[...] * cdif)
        y1_ref[...] = y1
        valid = _valid_mask(i)
        s1_ref[...] += jnp.sum(jnp.where(valid, y1, 0.0), axis=1, keepdims=True)
        s2_ref[...] += jnp.sum(jnp.where(valid, y1 * y1, 0.0), axis=1,
                               keepdims=True)

    return pl.pallas_call(
        kern,
        grid=(_GRID,),
        in_specs=[
            pl.BlockSpec((6, _BP), lambda i: (0, i)),
            pl.BlockSpec((6, _BN), lambda i: (0, i)),
            pl.BlockSpec((16, 6), lambda i: (0, 0)),
            pl.BlockSpec((16, 6), lambda i: (0, 0)),
            pl.BlockSpec((16, 1), lambda i: (0, 0)),
            pl.BlockSpec((16, 1), lambda i: (0, 0)),
        ],
        out_specs=[
            pl.BlockSpec((16, _BP), lambda i: (0, i)),
            pl.BlockSpec((16, 1), lambda i: (0, 0)),
            pl.BlockSpec((16, 1), lambda i: (0, 0)),
        ],
        out_shape=[
            jax.ShapeDtypeStruct((16, _GRID * _BP), jnp.float32),
            jax.ShapeDtypeStruct((16, 1), jnp.float32),
            jax.ShapeDtypeStruct((16, 1), jnp.float32),
        ],
        scratch_shapes=[pltpu.VMEM((_BN, _BP), jnp.float32)],
    )(nbT, coordsT, we, wn, wd, wc)


def _stats2(y1hbm, a1, c1):
    def kern(y1_ref, a1_ref, c1_ref, hs_ref, hh_ref):
        i = pl.program_id(0)

        @pl.when(i == 0)
        def _():
            hs_ref[...] = jnp.zeros_like(hs_ref)
            hh_ref[...] = jnp.zeros_like(hh_ref)

        h = jnp.maximum(a1_ref[...] * y1_ref[...] + c1_ref[...], 0.0)
        hm = jnp.where(_valid_mask(i), h, 0.0)
        hs_ref[...] += jnp.sum(hm, axis=1, keepdims=True)
        hh_ref[...] += lax.dot_general(hm, h, (((1,), (1,)), ((), ())),
                                       preferred_element_type=jnp.float32)

    return pl.pallas_call(
        kern,
        grid=(_GRID,),
        in_specs=[
            pl.BlockSpec((16, _BP), lambda i: (0, i)),
            pl.BlockSpec((16, 1), lambda i: (0, 0)),
            pl.BlockSpec((16, 1), lambda i: (0, 0)),
        ],
        out_specs=[
            pl.BlockSpec((16, 1), lambda i: (0, 0)),
            pl.BlockSpec((16, 16), lambda i: (0, 0)),
        ],
        out_shape=[
            jax.ShapeDtypeStruct((16, 1), jnp.float32),
            jax.ShapeDtypeStruct((16, 16), jnp.float32),
        ],
    )(y1hbm, a1, c1)


def _final(y1hbm, featsND, a1, c1, a2w, c2r):
    def kern(y1_ref, f_ref, a1_ref, c1_ref, a2w_ref, c2_ref, x_ref, ste_ref):
        h = jnp.maximum(a1_ref[...] * y1_ref[...] + c1_ref[...], 0.0)
        ot = jnp.maximum(
            _split3_dot(h, a2w_ref[...], (((0,), (1,)), ((), ())))
            + c2_ref[...], 0.0)
        ef = jnp.broadcast_to(f_ref[...][:, None, :],
                              (_BNF, _K, _D)).reshape(_BPF, _D)
        ste_ref[...] = ot
        x_ref[:, 0:_D] = ot
        x_ref[:, _D:2 * _D] = ef

    return pl.pallas_call(
        kern,
        grid=(_GRIDF,),
        in_specs=[
            pl.BlockSpec((16, _BPF), lambda i: (0, i)),
            pl.BlockSpec((_BNF, _D), lambda i: (i, 0)),
            pl.BlockSpec((16, 1), lambda i: (0, 0)),
            pl.BlockSpec((16, 1), lambda i: (0, 0)),
            pl.BlockSpec((_D, 16), lambda i: (0, 0)),
            pl.BlockSpec((1, _D), lambda i: (0, 0)),
        ],
        out_specs=[
            pl.BlockSpec((_BPF, 2 * _D), lambda i: (i, 0)),
            pl.BlockSpec((_BPF, _D), lambda i: (i, 0)),
        ],
        out_shape=[
            jax.ShapeDtypeStruct((_P, 2 * _D), jnp.float32),
            jax.ShapeDtypeStruct((_P, _D), jnp.float32),
        ],
    )(y1hbm, featsND, a1, c1, a2w, c2r)


def kernel(coords, raw_points, features, neigh_idx, whether_cal,
           W0, b0, g0, be0, W1, b1, g1, be1):
    f32 = jnp.float32
    idx = neigh_idx.astype(jnp.int32).reshape(_P)
    idx_pad = jnp.concatenate([idx, jnp.zeros((_PPAD - _P,), jnp.int32)])
    rawflat = raw_points.reshape(_M * 6).astype(f32)

    nbT = _sc_gather(rawflat, idx_pad)

    coordsT = jnp.transpose(coords.reshape(_N, 6).astype(f32))
    featsND = jnp.transpose(features.reshape(_D, _N).astype(f32))

    W0f = W0.astype(f32)
    we0 = W0f[:, 0:6] + W0f[:, 12:18]
    wn0 = W0f[:, 6:12] - W0f[:, 12:18]
    wd0 = W0f[:, 18:19]
    wc0 = W0f[:, 19:20]

    y1hbm, s1, s2 = _stats1(nbT, coordsT, we0, wn0, wd0, wc0)
    mean1 = s1[:, 0] / _P
    var1 = s2[:, 0] / _P - mean1 * mean1
    a1 = g0 * lax.rsqrt(var1 + _EPS)
    c1 = be0 - a1 * mean1

    hs, hh = _stats2(y1hbm, a1[:, None], c1[:, None])
    W1f = W1.astype(f32)
    mh = hs[:, 0] / _P
    HH = hh / _P
    mean2 = jnp.dot(W1f, mh, precision=lax.Precision.HIGHEST)
    wHH = jnp.dot(W1f, HH, precision=lax.Precision.HIGHEST)
    var2 = jnp.sum(wHH * W1f, axis=1) - mean2 * mean2
    a2 = g1 * lax.rsqrt(var2 + _EPS)
    c2 = be1 - a2 * mean2
    a2w = a2[:, None] * W1.astype(f32)

    x2, ste2 = _final(y1hbm, featsND, a1[:, None], c1[:, None],
                      a2w, c2[None, :])
    x = jnp.transpose(x2.reshape(1, _N, _K, 2 * _D), (0, 3, 1, 2))
    ste = jnp.transpose(ste2.reshape(1, _N, _K, _D), (0, 3, 1, 2))
    return (x, ste)


# speedup vs baseline: 1.0448x; 1.0278x over previous
import functools

import jax
import jax.numpy as jnp
from jax import lax
from jax.experimental import pallas as pl
from jax.experimental.pallas import tpu as pltpu
from jax.experimental.pallas import tpu_sc as plsc

_N = 10000
_M = 10000
_K = 16
_D = 128
_P = _N * _K
_NW = 32
_CHUNK = 5120
_PPAD = _NW * _CHUNK
_BP = 4096
_BN = _BP // _K
_GRID = -(-_P // _BP)
_BPF = 8192
_BNF = _BPF // _K
_GRIDF = -(-_P // _BPF)
_EPS = 1e-6
_MM = (((1,), (0,)), ((), ()))


def _sc_gather(raw2d, idx_pad):
    mesh = plsc.VectorSubcoreMesh(core_axis_name="c", subcore_axis_name="s")

    @functools.partial(
        pl.kernel,
        mesh=mesh,
        out_type=jax.ShapeDtypeStruct((6, _PPAD), jnp.float32),
        compiler_params=pltpu.CompilerParams(needs_layout_passes=False),
        scratch_types=[
            pltpu.VMEM((_M * 6,), jnp.float32),
            pltpu.VMEM((_CHUNK,), jnp.int32),
            pltpu.VMEM((6, _CHUNK), jnp.float32),
        ],
    )
    def gather_kernel(raw_hbm, idx_hbm, out_hbm, table_v, idx_v, nb_v):
        wid = lax.axis_index("s") * 2 + lax.axis_index("c")
        base = wid * _CHUNK
        pltpu.sync_copy(raw_hbm, table_v)
        pltpu.sync_copy(idx_hbm.at[pl.ds(base, _CHUNK)], idx_v)

        @plsc.parallel_loop(0, _CHUNK, 16, unroll=32)
        def _(off):
            rows = idx_v[pl.ds(off, 16)] * 6
            for ch in range(6):
                nb_v[ch, pl.ds(off, 16)] = plsc.load_gather(table_v, [rows + ch])
        pltpu.sync_copy(nb_v, out_hbm.at[:, pl.ds(base, _CHUNK)])

    return gather_kernel(raw2d, idx_pad)


def _valid_mask(i):
    lane = lax.broadcasted_iota(jnp.int32, (1, _BP), 1)
    return (i * _BP + lane) < _P


def _split_dot_onehot(x, e):
    hi = x.astype(jnp.bfloat16).astype(jnp.float32)
    lo = x - hi
    return (lax.dot_general(hi, e, _MM, preferred_element_type=jnp.float32)
            + lax.dot_general(lo, e, _MM, preferred_element_type=jnp.float32))


def _split3_dot(a, b, dims):
    ahi = a.astype(jnp.bfloat16).astype(jnp.float32)
    alo = a - ahi
    bhi = b.astype(jnp.bfloat16).astype(jnp.float32)
    blo = b - bhi

    def d(x, y):
        return lax.dot_general(x, y, dims, preferred_element_type=jnp.float32)

    return d(ahi, bhi) + d(ahi, blo) + d(alo, bhi)


def _stats1(nbT, coordsT, wen, wd, wc):
    def kern(nb_ref, ct_ref, wen_ref, wd_ref, wc_ref,
             y1_ref, s1_ref, s2_ref, e_ref):
        i = pl.program_id(0)

        @pl.when(i == 0)
        def _():
            r = lax.broadcasted_iota(jnp.int32, (_BN, _BP), 0)
            c = lax.broadcasted_iota(jnp.int32, (_BN, _BP), 1)
            e_ref[...] = (r == (c // _K)).astype(jnp.float32)
            s1_ref[...] = jnp.zeros_like(s1_ref)
            s2_ref[...] = jnp.zeros_like(s2_ref)

        lane_p = lax.broadcasted_iota(jnp.int32, (1, _BP), 1)
        nb = jnp.where(i * _BP + lane_p < _P, nb_ref[...], 0.0)
        lane_n = lax.broadcasted_iota(jnp.int32, (1, _BN), 1)
        ct = jnp.where(i * _BN + lane_n < _N, ct_ref[...], 0.0)

        ext = _split_dot_onehot(ct, e_ref[...])
        diff = ext - nb
        d3 = diff[0:3, :]
        c3 = diff[3:6, :]
        dist = jnp.sqrt(jnp.sum(d3 * d3, axis=0, keepdims=True))
        cdif = jnp.sqrt(jnp.sum(c3 * c3, axis=0, keepdims=True))
        cat = jnp.concatenate([ext, nb], axis=0)
        y1 = (_split3_dot(wen_ref[...], cat, _MM)
              + wd_ref[...] * dist + wc_ref[...] * cdif)
        y1_ref[...] = y1
        valid = _valid_mask(i)
        s1_ref[...] += jnp.sum(jnp.where(valid, y1, 0.0), axis=1, keepdims=True)
        s2_ref[...] += jnp.sum(jnp.where(valid, y1 * y1, 0.0), axis=1,
                               keepdims=True)

    return pl.pallas_call(
        kern,
        grid=(_GRID,),
        in_specs=[
            pl.BlockSpec((6, _BP), lambda i: (0, i)),
            pl.BlockSpec((6, _BN), lambda i: (0, i)),
            pl.BlockSpec((16, 12), lambda i: (0, 0)),
            pl.BlockSpec((16, 1), lambda i: (0, 0)),
            pl.BlockSpec((16, 1), lambda i: (0, 0)),
        ],
        out_specs=[
            pl.BlockSpec((16, _BP), lambda i: (0, i)),
            pl.BlockSpec((16, 1), lambda i: (0, 0)),
            pl.BlockSpec((16, 1), lambda i: (0, 0)),
        ],
        out_shape=[
            jax.ShapeDtypeStruct((16, _GRID * _BP), jnp.float32),
            jax.ShapeDtypeStruct((16, 1), jnp.float32),
            jax.ShapeDtypeStruct((16, 1), jnp.float32),
        ],
        scratch_shapes=[pltpu.VMEM((_BN, _BP), jnp.float32)],
    )(nbT, coordsT, wen, wd, wc)


def _stats2(y1hbm, a1, c1):
    def kern(y1_ref, a1_ref, c1_ref, hs_ref, hh_ref):
        i = pl.program_id(0)

        @pl.when(i == 0)
        def _():
            hs_ref[...] = jnp.zeros_like(hs_ref)
            hh_ref[...] = jnp.zeros_like(hh_ref)

        h = jnp.maximum(a1_ref[...] * y1_ref[...] + c1_ref[...], 0.0)
        hm = jnp.where(_valid_mask(i), h, 0.0)
        hs_ref[...] += jnp.sum(hm, axis=1, keepdims=True)
        hh_ref[...] += lax.dot_general(hm, h, (((1,), (1,)), ((), ())),
                                       preferred_element_type=jnp.float32)

    return pl.pallas_call(
        kern,
        grid=(_GRID,),
        in_specs=[
            pl.BlockSpec((16, _BP), lambda i: (0, i)),
            pl.BlockSpec((16, 1), lambda i: (0, 0)),
            pl.BlockSpec((16, 1), lambda i: (0, 0)),
        ],
        out_specs=[
            pl.BlockSpec((16, 1), lambda i: (0, 0)),
            pl.BlockSpec((16, 16), lambda i: (0, 0)),
        ],
        out_shape=[
            jax.ShapeDtypeStruct((16, 1), jnp.float32),
            jax.ShapeDtypeStruct((16, 16), jnp.float32),
        ],
    )(y1hbm, a1, c1)


def _final(y1hbm, featsND, a1, c1, a2w, c2r):
    def kern(y1_ref, f_ref, a1_ref, c1_ref, a2w_ref, c2_ref, x_ref, ste_ref):
        h = jnp.maximum(a1_ref[...] * y1_ref[...] + c1_ref[...], 0.0)
        ot = jnp.maximum(
            _split3_dot(h, a2w_ref[...], (((0,), (1,)), ((), ())))
            + c2_ref[...], 0.0)
        ef = jnp.broadcast_to(f_ref[...][:, None, :],
                              (_BNF, _K, _D)).reshape(_BPF, _D)
        ste_ref[...] = ot
        x_ref[:, 0:_D] = ot
        x_ref[:, _D:2 * _D] = ef

    return pl.pallas_call(
        kern,
        grid=(_GRIDF,),
        in_specs=[
            pl.BlockSpec((16, _BPF), lambda i: (0, i)),
            pl.BlockSpec((_BNF, _D), lambda i: (i, 0)),
            pl.BlockSpec((16, 1), lambda i: (0, 0)),
            pl.BlockSpec((16, 1), lambda i: (0, 0)),
            pl.BlockSpec((_D, 16), lambda i: (0, 0)),
            pl.BlockSpec((1, _D), lambda i: (0, 0)),
        ],
        out_specs=[
            pl.BlockSpec((_BPF, 2 * _D), lambda i: (i, 0)),
            pl.BlockSpec((_BPF, _D), lambda i: (i, 0)),
        ],
        out_shape=[
            jax.ShapeDtypeStruct((_P, 2 * _D), jnp.float32),
            jax.ShapeDtypeStruct((_P, _D), jnp.float32),
        ],
    )(y1hbm, featsND, a1, c1, a2w, c2r)


def kernel(coords, raw_points, features, neigh_idx, whether_cal,
           W0, b0, g0, be0, W1, b1, g1, be1):
    f32 = jnp.float32
    idx = neigh_idx.astype(jnp.int32).reshape(_P)
    idx_pad = jnp.concatenate([idx, jnp.zeros((_PPAD - _P,), jnp.int32)])
    rawflat = raw_points.reshape(_M * 6).astype(f32)

    nbT = _sc_gather(rawflat, idx_pad)

    coordsT = jnp.transpose(coords.reshape(_N, 6).astype(f32))
    featsND = jnp.transpose(features.reshape(_D, _N).astype(f32))

    W0f = W0.astype(f32)
    we0 = W0f[:, 0:6] + W0f[:, 12:18]
    wn0 = W0f[:, 6:12] - W0f[:, 12:18]
    wd0 = W0f[:, 18:19]
    wc0 = W0f[:, 19:20]

    y1hbm, s1, s2 = _stats1(nbT, coordsT,
                            jnp.concatenate([we0, wn0], axis=1), wd0, wc0)
    mean1 = s1[:, 0] / _P
    var1 = s2[:, 0] / _P - mean1 * mean1
    a1 = g0 * lax.rsqrt(var1 + _EPS)
    c1 = be0 - a1 * mean1

    hs, hh = _stats2(y1hbm, a1[:, None], c1[:, None])
    W1f = W1.astype(f32)
    mh = hs[:, 0] / _P
    HH = hh / _P
    mean2 = jnp.dot(W1f, mh, precision=lax.Precision.HIGHEST)
    wHH = jnp.dot(W1f, HH, precision=lax.Precision.HIGHEST)
    var2 = jnp.sum(wHH * W1f, axis=1) - mean2 * mean2
    a2 = g1 * lax.rsqrt(var2 + _EPS)
    c2 = be1 - a2 * mean2
    a2w = a2[:, None] * W1.astype(f32)

    x2, ste2 = _final(y1hbm, featsND, a1[:, None], c1[:, None],
                      a2w, c2[None, :])
    x = jnp.transpose(x2.reshape(1, _N, _K, 2 * _D), (0, 3, 1, 2))
    ste = jnp.transpose(ste2.reshape(1, _N, _K, _D), (0, 3, 1, 2))
    return (x, ste)
